# Initial kernel scaffold; baseline (speedup 1.0000x reference)
import jax
import jax.numpy as jnp
from jax.experimental import pallas as pl


def kernel(x, A, B):
    raise NotImplementedError("write your pallas kernel here")


# speedup vs baseline: 1.5536x; 1.5536x over previous
import functools

import jax
import jax.numpy as jnp
from jax import lax
from jax.experimental import pallas as pl
from jax.experimental.pallas import tpu as pltpu
from jax.experimental.pallas import tpu_sc as plsc

_NUM_EMBEDDINGS = 1000000
_EMBEDDING_DIM = 64
_R = 4
_SCALING = 1.0 / _R
_ROW = 8

_NW = 32
_CHUNK = 128


def _adapter_kernel(n_per_w, n_chunks, x_hbm, bt_hbm, t_hbm, out_hbm,
                    idx_v, row_v, bt_v, out_v, sem):
    wid = lax.axis_index("s") * 2 + lax.axis_index("c")
    base = wid * n_per_w

    pltpu.sync_copy(bt_hbm, bt_v)
    bt = [[bt_v[r, pl.ds(db * 16, 16)] for db in range(4)] for r in range(_R)]

    def chunk_body(c, carry):
        off = base + c * _CHUNK
        pltpu.sync_copy(x_hbm.at[pl.ds(off, _CHUNK)], idx_v)
        pltpu.make_async_copy(t_hbm.at[idx_v], row_v, sem).start()
        pltpu.make_async_copy(t_hbm.at[idx_v], row_v, sem).wait()

        def n_body(n, carry2):
            nsplat = jnp.full((16,), n, dtype=jnp.int32)
            e = [plsc.load_gather(
                     row_v, [nsplat, jnp.full((16,), r, dtype=jnp.int32)])
                 for r in range(_R)]
            for db in range(4):
                acc = e[0] * bt[0][db]
                acc = acc + e[1] * bt[1][db]
                acc = acc + e[2] * bt[2][db]
                acc = acc + e[3] * bt[3][db]
                out_v[pl.ds(n * _EMBEDDING_DIM + db * 16, 16)] = acc
            return carry2

        lax.fori_loop(0, _CHUNK, n_body, 0, unroll=2)
        pltpu.sync_copy(out_v, out_hbm.at[pl.ds(off * _EMBEDDING_DIM,
                                                _CHUNK * _EMBEDDING_DIM)])
        return carry

    lax.fori_loop(0, n_chunks, chunk_body, 0)


def kernel(x, A, B):
    n = x.shape[0] * x.shape[1]
    xf = x.reshape(n).astype(jnp.int32)
    bt = (B.T * _SCALING).astype(jnp.float32)
    table = jnp.pad(A.T.astype(jnp.float32), ((0, 0), (0, _ROW - _R)))
    n_per_w = n // _NW
    n_chunks = n_per_w // _CHUNK

    mesh = plsc.VectorSubcoreMesh(core_axis_name="c", subcore_axis_name="s")
    run = pl.kernel(
        functools.partial(_adapter_kernel, n_per_w, n_chunks),
        out_type=jax.ShapeDtypeStruct((n * _EMBEDDING_DIM,), jnp.float32),
        mesh=mesh,
        compiler_params=pltpu.CompilerParams(
            needs_layout_passes=False, use_tc_tiling_on_sc=False),
        scratch_types=[
            pltpu.VMEM((_CHUNK,), jnp.int32),
            pltpu.VMEM((_CHUNK, _ROW), jnp.float32),
            pltpu.VMEM((_R, _EMBEDDING_DIM), jnp.float32),
            pltpu.VMEM((_CHUNK * _EMBEDDING_DIM,), jnp.float32),
            pltpu.SemaphoreType.DMA,
        ],
    )
    out = run(xf, bt, table)
    return out.reshape(x.shape[0], x.shape[1], _EMBEDDING_DIM)


# speedup vs baseline: 1.5817x; 1.0181x over previous
import functools

import jax
import jax.numpy as jnp
from jax import lax
from jax.experimental import pallas as pl
from jax.experimental.pallas import tpu as pltpu
from jax.experimental.pallas import tpu_sc as plsc

_NUM_EMBEDDINGS = 1000000
_EMBEDDING_DIM = 64
_R = 4
_SCALING = 1.0 / _R
_ROW = 8

_NW = 32
_CHUNK = 128


def _adapter_kernel(n_per_w, n_chunks, x_hbm, bt_hbm, t_hbm, out_hbm,
                    idx_v, row_v, bt_v, out_v, sem):
    wid = lax.axis_index("s") * 2 + lax.axis_index("c")
    base = wid * n_per_w

    pltpu.sync_copy(bt_hbm, bt_v)
    bt = [[bt_v[r, pl.ds(db * 16, 16)] for db in range(4)] for r in range(_R)]

    def chunk_body(c, carry):
        off = base + c * _CHUNK
        pltpu.sync_copy(x_hbm.at[pl.ds(off, _CHUNK)], idx_v)
        pltpu.make_async_copy(t_hbm.at[idx_v], row_v, sem).start()
        pltpu.make_async_copy(t_hbm.at[idx_v], row_v, sem).wait()

        @plsc.parallel_loop(0, _CHUNK, unroll=4)
        def n_body(n):
            nsplat = jnp.full((16,), n, dtype=jnp.int32)
            e = [plsc.load_gather(
                     row_v, [nsplat, jnp.full((16,), r, dtype=jnp.int32)])
                 for r in range(_R)]
            for db in range(4):
                acc = e[0] * bt[0][db]
                acc = acc + e[1] * bt[1][db]
                acc = acc + e[2] * bt[2][db]
                acc = acc + e[3] * bt[3][db]
                out_v[pl.ds(n * _EMBEDDING_DIM + db * 16, 16)] = acc
        pltpu.sync_copy(out_v, out_hbm.at[pl.ds(off * _EMBEDDING_DIM,
                                                _CHUNK * _EMBEDDING_DIM)])
        return carry

    lax.fori_loop(0, n_chunks, chunk_body, 0)


def kernel(x, A, B):
    n = x.shape[0] * x.shape[1]
    xf = x.reshape(n).astype(jnp.int32)
    bt = (B.T * _SCALING).astype(jnp.float32)
    table = jnp.pad(A.T.astype(jnp.float32), ((0, 0), (0, _ROW - _R)))
    n_per_w = n // _NW
    n_chunks = n_per_w // _CHUNK

    mesh = plsc.VectorSubcoreMesh(core_axis_name="c", subcore_axis_name="s")
    run = pl.kernel(
        functools.partial(_adapter_kernel, n_per_w, n_chunks),
        out_type=jax.ShapeDtypeStruct((n * _EMBEDDING_DIM,), jnp.float32),
        mesh=mesh,
        compiler_params=pltpu.CompilerParams(
            needs_layout_passes=False, use_tc_tiling_on_sc=False),
        scratch_types=[
            pltpu.VMEM((_CHUNK,), jnp.int32),
            pltpu.VMEM((_CHUNK, _ROW), jnp.float32),
            pltpu.VMEM((_R, _EMBEDDING_DIM), jnp.float32),
            pltpu.VMEM((_CHUNK * _EMBEDDING_DIM,), jnp.float32),
            pltpu.SemaphoreType.DMA,
        ],
    )
    out = run(xf, bt, table)
    return out.reshape(x.shape[0], x.shape[1], _EMBEDDING_DIM)


# speedup vs baseline: 8.5367x; 5.3973x over previous
import functools

import jax
import jax.numpy as jnp
from jax import lax
from jax.experimental import pallas as pl
from jax.experimental.pallas import tpu as pltpu
from jax.experimental.pallas import tpu_sc as plsc

_NUM_EMBEDDINGS = 1000000
_EMBEDDING_DIM = 64
_R = 4
_SCALING = 1.0 / _R
_ROW = 8

_NW = 32
_CHUNK = 128


def _adapter_kernel(n_per_w, n_chunks, x_hbm, bt_hbm, a_hbm, out_hbm,
                    idx_v, idx8_v, r0_v, r1_v, r2_v, r3_v, bt_v, out_v, sem):
    wid = lax.axis_index("s") * 2 + lax.axis_index("c")
    base = wid * n_per_w
    row_refs = [r0_v, r1_v, r2_v, r3_v]

    pltpu.sync_copy(bt_hbm, bt_v)
    bt = [[bt_v[r, pl.ds(db * 16, 16)] for db in range(4)] for r in range(_R)]

    def chunk_body(c, carry):
        off = base + c * _CHUNK
        pltpu.sync_copy(x_hbm.at[pl.ds(off, _CHUNK)], idx_v)
        for v in range(_CHUNK // 16):
            idx8_v[pl.ds(v * 16, 16)] = lax.shift_right_logical(
                idx_v[pl.ds(v * 16, 16)], 3)
        copies = [
            pltpu.make_async_copy(a_hbm.at[r].at[idx8_v], row_refs[r], sem)
            for r in range(_R)
        ]
        for cp in copies:
            cp.start()
        for cp in copies:
            cp.wait()

        @plsc.parallel_loop(0, _CHUNK, unroll=4)
        def n_body(n):
            nsplat = jnp.full((16,), n, dtype=jnp.int32)
            ivec = plsc.load_gather(idx_v, [nsplat])
            lane = lax.bitwise_and(ivec, jnp.full((16,), 7, dtype=jnp.int32))
            e = [plsc.load_gather(row_refs[r], [nsplat, lane])
                 for r in range(_R)]
            for db in range(4):
                acc = e[0] * bt[0][db]
                acc = acc + e[1] * bt[1][db]
                acc = acc + e[2] * bt[2][db]
                acc = acc + e[3] * bt[3][db]
                out_v[pl.ds(n * _EMBEDDING_DIM + db * 16, 16)] = acc

        pltpu.sync_copy(out_v, out_hbm.at[pl.ds(off * _EMBEDDING_DIM,
                                                _CHUNK * _EMBEDDING_DIM)])
        return carry

    lax.fori_loop(0, n_chunks, chunk_body, 0)


def kernel(x, A, B):
    n = x.shape[0] * x.shape[1]
    xf = x.reshape(n).astype(jnp.int32)
    bt = (B.T * _SCALING).astype(jnp.float32)
    table = A.reshape(_R, _NUM_EMBEDDINGS // _ROW, _ROW)
    n_per_w = n // _NW
    n_chunks = n_per_w // _CHUNK

    mesh = plsc.VectorSubcoreMesh(core_axis_name="c", subcore_axis_name="s")
    run = pl.kernel(
        functools.partial(_adapter_kernel, n_per_w, n_chunks),
        out_type=jax.ShapeDtypeStruct((n * _EMBEDDING_DIM,), jnp.float32),
        mesh=mesh,
        compiler_params=pltpu.CompilerParams(
            needs_layout_passes=False, use_tc_tiling_on_sc=False),
        scratch_types=[
            pltpu.VMEM((_CHUNK,), jnp.int32),
            pltpu.VMEM((_CHUNK,), jnp.int32),
            pltpu.VMEM((_CHUNK, _ROW), jnp.float32),
            pltpu.VMEM((_CHUNK, _ROW), jnp.float32),
            pltpu.VMEM((_CHUNK, _ROW), jnp.float32),
            pltpu.VMEM((_CHUNK, _ROW), jnp.float32),
            pltpu.VMEM((_R, _EMBEDDING_DIM), jnp.float32),
            pltpu.VMEM((_CHUNK * _EMBEDDING_DIM,), jnp.float32),
            pltpu.SemaphoreType.DMA,
        ],
    )
    out = run(xf, bt, table)
    return out.reshape(x.shape[0], x.shape[1], _EMBEDDING_DIM)


# speedup vs baseline: 19.0540x; 2.2320x over previous
import functools

import jax
import jax.numpy as jnp
from jax import lax
from jax.experimental import pallas as pl
from jax.experimental.pallas import tpu as pltpu
from jax.experimental.pallas import tpu_sc as plsc

_NUM_EMBEDDINGS = 1000000
_D = 64
_R = 4
_SCALING = 1.0 / _R
_ROW = 8

_NW = 32
_B = 4096
_L = 50
_BW = _B // _NW
_LC = 5
_NBB = _BW // 16


def _adapter_kernel(x_hbm, bt_hbm, a_hbm, out_hbm,
                    xs_v, idx8_v, lane_v, r0_v, r1_v, r2_v, r3_v, bt_v,
                    out_v, sem):
    wid = lax.axis_index("s") * 2 + lax.axis_index("c")
    row_refs = [r0_v, r1_v, r2_v, r3_v]

    pltpu.sync_copy(x_hbm.at[pl.ds(wid * (_BW * _L), _BW * _L)], xs_v)
    pltpu.sync_copy(bt_hbm, bt_v)

    i50 = jax.lax.iota(jnp.int32, 16) * _L
    bvecs = [jax.lax.iota(jnp.int32, 16) + bb * 16 for bb in range(_NBB)]
    seven = jnp.full((16,), 7, dtype=jnp.int32)
    rsplat = [jnp.full((16,), r, dtype=jnp.int32) for r in range(_R)]

    def chunk_body(c, carry):
        l0 = c * _LC
        l0splat = jnp.full((16,), l0, dtype=jnp.int32)
        for lp in range(_LC):
            for bb in range(_NBB):
                pos = i50 + (bb * (16 * _L) + lp)
                iv = plsc.load_gather(xs_v, [pos + l0splat])
                idx8_v[lp, pl.ds(bb * 16, 16)] = lax.shift_right_logical(
                    iv, 3)
                lane_v[lp, pl.ds(bb * 16, 16)] = lax.bitwise_and(iv, seven)
        copies = []
        for lp in range(_LC):
            for r in range(_R):
                copies.append(pltpu.make_async_copy(
                    a_hbm.at[r].at[idx8_v.at[lp]], row_refs[r].at[lp], sem))
        for cp in copies:
            cp.start()
        for cp in copies:
            cp.wait()

        for lp in range(_LC):
            lpsplat = jnp.full((16,), lp, dtype=jnp.int32)
            lanes = [lane_v[lp, pl.ds(bb * 16, 16)] for bb in range(_NBB)]
            e = [[plsc.load_gather(row_refs[r], [lpsplat, bvecs[bb],
                                                 lanes[bb]])
                  for bb in range(_NBB)] for r in range(_R)]

            @plsc.parallel_loop(0, _D, unroll=2)
            def d_body(d, _lp=lp, _e=e):
                dsplat = jnp.full((16,), d, dtype=jnp.int32)
                s = [plsc.load_gather(bt_v, [rsplat[r], dsplat])
                     for r in range(_R)]
                dt = lax.shift_right_logical(d, 3)
                di = lax.bitwise_and(d, 7)
                for bb in range(_NBB):
                    acc = _e[0][bb] * s[0]
                    acc = acc + _e[1][bb] * s[1]
                    acc = acc + _e[2][bb] * s[2]
                    acc = acc + _e[3][bb] * s[3]
                    out_v[_lp, dt, di, pl.ds(bb * 16, 16)] = acc

        pltpu.sync_copy(out_v, out_hbm.at[pl.ds(l0, _LC), :, wid, :, :])
        return carry

    lax.fori_loop(0, _L // _LC, chunk_body, 0)


def kernel(x, A, B):
    xf = x.reshape(_B * _L).astype(jnp.int32)
    bt = (B.T * _SCALING).astype(jnp.float32)
    table = A.reshape(_R, _NUM_EMBEDDINGS // _ROW, _ROW)

    mesh = plsc.VectorSubcoreMesh(core_axis_name="c", subcore_axis_name="s")
    run = pl.kernel(
        _adapter_kernel,
        out_type=jax.ShapeDtypeStruct((_L, _D // 8, _NW, 8, _BW),
                                      jnp.float32),
        mesh=mesh,
        compiler_params=pltpu.CompilerParams(
            needs_layout_passes=False, use_tc_tiling_on_sc=False),
        scratch_types=[
            pltpu.VMEM((_BW * _L,), jnp.int32),
            pltpu.VMEM((_LC, _BW), jnp.int32),
            pltpu.VMEM((_LC, _BW), jnp.int32),
            pltpu.VMEM((_LC, _BW, _ROW), jnp.float32),
            pltpu.VMEM((_LC, _BW, _ROW), jnp.float32),
            pltpu.VMEM((_LC, _BW, _ROW), jnp.float32),
            pltpu.VMEM((_LC, _BW, _ROW), jnp.float32),
            pltpu.VMEM((_R, _D), jnp.float32),
            pltpu.VMEM((_LC, _D // 8, 8, _BW), jnp.float32),
            pltpu.SemaphoreType.DMA,
        ],
    )
    out5 = run(xf, bt, table)
    return out5.transpose(2, 4, 0, 1, 3).reshape(_B, _L, _D)


# speedup vs baseline: 23.5411x; 1.2355x over previous
import jax
import jax.numpy as jnp
from jax import lax
from jax.experimental import pallas as pl
from jax.experimental.pallas import tpu as pltpu
from jax.experimental.pallas import tpu_sc as plsc

_NUM_EMBEDDINGS = 1000000
_D = 64
_R = 4
_SCALING = 1.0 / _R
_ROW = 8

_NW = 32
_B = 4096
_L = 50
_BW = _B // _NW
_LC = 5
_NC = _L // _LC
_NBB = _BW // 16


def _adapter_kernel(x_hbm, bt_hbm, a_hbm, out_hbm,
                    xs_v, i8a, lna, i8b, lnb,
                    ra0, ra1, ra2, ra3, rb0, rb1, rb2, rb3,
                    bt_v, out_v, gsa, gsb, osem):
    wid = lax.axis_index("s") * 2 + lax.axis_index("c")
    rows_a = [ra0, ra1, ra2, ra3]
    rows_b = [rb0, rb1, rb2, rb3]

    pltpu.sync_copy(x_hbm.at[pl.ds(wid * (_BW * _L), _BW * _L)], xs_v)
    pltpu.sync_copy(bt_hbm, bt_v)

    i50 = jax.lax.iota(jnp.int32, 16) * _L
    bvecs = [jax.lax.iota(jnp.int32, 16) + bb * 16 for bb in range(_NBB)]
    seven = jnp.full((16,), 7, dtype=jnp.int32)
    rsplat = [jnp.full((16,), r, dtype=jnp.int32) for r in range(_R)]

    def build_lists(c, i8, ln):
        l0splat = jnp.full((16,), c * _LC, dtype=jnp.int32)
        for lp in range(_LC):
            for bb in range(_NBB):
                pos = i50 + (bb * (16 * _L) + lp)
                iv = plsc.load_gather(xs_v, [pos + l0splat])
                i8[lp, pl.ds(bb * 16, 16)] = lax.shift_right_logical(iv, 3)
                ln[lp, pl.ds(bb * 16, 16)] = lax.bitwise_and(iv, seven)

    def gather_copies(i8, rows, sem):
        return [pltpu.make_async_copy(a_hbm.at[r].at[i8.at[lp]],
                                      rows[r].at[lp], sem)
                for lp in range(_LC) for r in range(_R)]

    def out_copy(c):
        return pltpu.make_async_copy(
            out_v, out_hbm.at[pl.ds(c * _LC, _LC), :, wid, :, :], osem)

    def compute(c, ln, rows):
        for lp in range(_LC):
            lpsplat = jnp.full((16,), lp, dtype=jnp.int32)
            lanes = [ln[lp, pl.ds(bb * 16, 16)] for bb in range(_NBB)]
            e = [[plsc.load_gather(rows[r], [lpsplat, bvecs[bb], lanes[bb]])
                  for bb in range(_NBB)] for r in range(_R)]

            @plsc.parallel_loop(0, _D, unroll=2)
            def d_body(d, _lp=lp, _e=e):
                dsplat = jnp.full((16,), d, dtype=jnp.int32)
                s = [plsc.load_gather(bt_v, [rsplat[r], dsplat])
                     for r in range(_R)]
                dt = lax.shift_right_logical(d, 3)
                di = lax.bitwise_and(d, 7)
                for bb in range(_NBB):
                    acc = _e[0][bb] * s[0]
                    acc = acc + _e[1][bb] * s[1]
                    acc = acc + _e[2][bb] * s[2]
                    acc = acc + _e[3][bb] * s[3]
                    out_v[_lp, dt, di, pl.ds(bb * 16, 16)] = acc
        out_copy(c).start()

    build_lists(0, i8a, lna)
    for cp in gather_copies(i8a, rows_a, gsa):
        cp.start()

    def pair_body(i, carry):
        c0 = 2 * i
        c1 = 2 * i + 1
        c2 = lax.min(c1 + 1, _NC - 1)
        build_lists(c1, i8b, lnb)
        for cp in gather_copies(i8b, rows_b, gsb):
            cp.start()
        for cp in gather_copies(i8a, rows_a, gsa):
            cp.wait()

        @pl.when(i > 0)
        def _():
            out_copy(c0 - 1).wait()
        compute(c0, lna, rows_a)

        build_lists(c2, i8a, lna)
        for cp in gather_copies(i8a, rows_a, gsa):
            cp.start()
        for cp in gather_copies(i8b, rows_b, gsb):
            cp.wait()
        out_copy(c0).wait()
        compute(c1, lnb, rows_b)
        return carry

    lax.fori_loop(0, _NC // 2, pair_body, 0)
    for cp in gather_copies(i8a, rows_a, gsa):
        cp.wait()
    out_copy(_NC - 1).wait()


def kernel(x, A, B):
    xf = x.reshape(_B * _L).astype(jnp.int32)
    bt = (B.T * _SCALING).astype(jnp.float32)
    table = A.reshape(_R, _NUM_EMBEDDINGS // _ROW, _ROW)

    mesh = plsc.VectorSubcoreMesh(core_axis_name="c", subcore_axis_name="s")
    run = pl.kernel(
        _adapter_kernel,
        out_type=jax.ShapeDtypeStruct((_L, _D // 8, _NW, 8, _BW),
                                      jnp.float32),
        mesh=mesh,
        compiler_params=pltpu.CompilerParams(
            needs_layout_passes=False, use_tc_tiling_on_sc=False),
        scratch_types=[
            pltpu.VMEM((_BW * _L,), jnp.int32),
            pltpu.VMEM((_LC, _BW), jnp.int32),
            pltpu.VMEM((_LC, _BW), jnp.int32),
            pltpu.VMEM((_LC, _BW), jnp.int32),
            pltpu.VMEM((_LC, _BW), jnp.int32),
            pltpu.VMEM((_LC, _BW, _ROW), jnp.float32),
            pltpu.VMEM((_LC, _BW, _ROW), jnp.float32),
            pltpu.VMEM((_LC, _BW, _ROW), jnp.float32),
            pltpu.VMEM((_LC, _BW, _ROW), jnp.float32),
            pltpu.VMEM((_LC, _BW, _ROW), jnp.float32),
            pltpu.VMEM((_LC, _BW, _ROW), jnp.float32),
            pltpu.VMEM((_LC, _BW, _ROW), jnp.float32),
            pltpu.VMEM((_LC, _BW, _ROW), jnp.float32),
            pltpu.VMEM((_R, _D), jnp.float32),
            pltpu.VMEM((_LC, _D // 8, 8, _BW), jnp.float32),
            pltpu.SemaphoreType.DMA,
            pltpu.SemaphoreType.DMA,
            pltpu.SemaphoreType.DMA,
        ],
    )
    out5 = run(xf, bt, table)
    return out5.transpose(2, 4, 0, 1, 3).reshape(_B, _L, _D)
